# baseline (speedup 1.0000x reference)
import functools

import jax
import jax.numpy as jnp
from jax import lax
from jax.experimental import pallas as pl
from jax.experimental.pallas import tpu as pltpu
from jax.experimental.pallas import tpu_sc as plsc

_NUM_CODES = 1024
_DIM = 64
_ROWS = 9216
_HALF = _ROWS // 2
_BLOCK = 2304
_HGRID = _HALF // _BLOCK
_COMMIT_W = 0.25

_SC_CORES = 2
_SC_SUBCORES = 16
_SC_WORKERS = _SC_CORES * _SC_SUBCORES
_ROWS_PER_W = _HALF // _SC_WORKERS
_GATHER_CHUNKS = 2
_CHUNK = _ROWS_PER_W // _GATHER_CHUNKS

_INTERPRET = False


def _vq_core(zb, cb, w):
    znorm = jnp.sum(zb * zb, axis=1, keepdims=True)
    cnorm = jnp.sum(cb * cb, axis=1)[None, :]
    cross = lax.dot_general(zb, cb, (((1,), (1,)), ((), ())))
    d = znorm - 2.0 * cross + cnorm

    dmin = jnp.min(d, axis=1, keepdims=True)
    mask = d == dmin
    maskf = mask.astype(jnp.float32)

    mm = lax.dot_general(maskf, w, (((1,), (0,)), ((), ())))
    idx_f = mm[:, 0:1] * 32.0 + mm[:, 1:2]
    anytie = jnp.max(mm[:, 2:3]) > 1.5

    e = jnp.exp(dmin - d)
    ones_n = jnp.ones((_NUM_CODES, 1), jnp.float32)
    s = lax.dot_general(e, ones_n, (((1,), (0,)), ((), ())))
    r = 1.0 / s
    block_soft = lax.dot_general(r, e, (((0,), (0,)), ((), ())))
    ones_b = jnp.ones((zb.shape[0], 1), jnp.float32)
    block_counts = lax.dot_general(ones_b, maskf,
                                   (((0,), (0,)), ((), ())))

    block_msum = jnp.sum(dmin, axis=0, keepdims=True)
    return d, mask, maskf, idx_f, anytie, block_counts, block_soft, block_msum


def _tie_fix(d, mask, maskf, idx_ref, cnt_accum_ref):
    iota = lax.broadcasted_iota(jnp.int32, d.shape, 1)
    idx_e = jnp.min(jnp.where(mask, iota, _NUM_CODES), axis=1)
    idx_ref[...] = idx_e.reshape(1, 1, -1)
    onehot = (iota == idx_e[:, None]).astype(jnp.float32)
    cnt_accum_ref[...] += jnp.sum(onehot - maskf, axis=0)[None, :]


def _vq_body_first(z_ref, cb_ref, w_ref,
                   idx_ref, cbpad_ref, cnt_ref, soft_ref, msum_ref):
    step = pl.program_id(0)
    cb = cb_ref[...]
    d, mask, maskf, idx_f, anytie, bc, bs, bm = _vq_core(z_ref[...], cb,
                                                         w_ref[...])
    idx_ref[...] = idx_f.astype(jnp.int32).reshape(1, 1, -1)

    @pl.when(step == 0)
    def _init():
        cbpad_ref[...] = jnp.concatenate([cb, cb], axis=1)
        cnt_ref[...] = bc
        soft_ref[...] = bs
        msum_ref[...] = bm

    @pl.when(step > 0)
    def _accum():
        cnt_ref[...] += bc
        soft_ref[...] += bs
        msum_ref[...] += bm

    @pl.when(anytie)
    def _fix():
        _tie_fix(d, mask, maskf, idx_ref, cnt_ref)


def _vq_body_second(z_ref, cb_ref, w_ref, pc_ref, ps_ref, pm_ref,
                    idx_ref, usage_ref, soft_ref,
                    loss_ref, cbl_ref, cml_ref, perp_ref):
    step = pl.program_id(0)
    d, mask, maskf, idx_f, anytie, bc, bs, bm = _vq_core(z_ref[...],
                                                         cb_ref[...],
                                                         w_ref[...])
    idx_ref[...] = idx_f.astype(jnp.int32).reshape(1, 1, -1)

    @pl.when(step == 0)
    def _init():
        usage_ref[...] = pc_ref[...] + bc
        soft_ref[...] = ps_ref[...] + bs
        loss_ref[...] = pm_ref[...] + bm

    @pl.when(step > 0)
    def _accum():
        usage_ref[...] += bc
        soft_ref[...] += bs
        loss_ref[...] += bm

    @pl.when(anytie)
    def _fix():
        _tie_fix(d, mask, maskf, idx_ref, usage_ref)

    @pl.when(step == _HGRID - 1)
    def _finalize():
        counts = usage_ref[...]
        total = jnp.sum(counts, axis=1, keepdims=True)
        u = counts / jnp.maximum(total, 1.0)
        usage_ref[...] = u
        soft_ref[...] = soft_ref[...] / float(_ROWS)
        mse = loss_ref[...] / float(_ROWS * _DIM)
        cbl_ref[...] = mse
        cml_ref[...] = mse
        loss_ref[...] = mse + _COMMIT_W * mse
        ent = jnp.sum(u * jnp.log(u + 1e-08), axis=1, keepdims=True)
        perp_ref[...] = jnp.exp(-ent)


def _sc_gather_body(cbpad_hbm, idx_hbm, out_hbm, idx_v, rows_v, comp_v, sem):
    wid = lax.axis_index("s") * _SC_CORES + lax.axis_index("c")
    base = wid * _ROWS_PER_W
    pltpu.sync_copy(idx_hbm.at[wid], idx_v)
    copies = [
        pltpu.async_copy(cbpad_hbm.at[idx_v.at[k]],
                         rows_v.at[pl.ds(k * _CHUNK, _CHUNK)], sem)
        for k in range(_GATHER_CHUNKS)
    ]
    for c in copies:
        c.wait()

    def _row(r, carry):
        for j in range(_DIM // 16):
            comp_v[r, pl.ds(16 * j, 16)] = rows_v[r, pl.ds(16 * j, 16)]
        return carry

    lax.fori_loop(0, _ROWS_PER_W, _row, 0, unroll=2)
    pltpu.sync_copy(comp_v, out_hbm.at[pl.ds(base, _ROWS_PER_W)])


def _sc_gather(cbpad, idx_w):
    gather = functools.partial(
        pl.kernel,
        mesh=plsc.VectorSubcoreMesh(core_axis_name="c", subcore_axis_name="s"),
        out_type=jax.ShapeDtypeStruct((_HALF, _DIM), jnp.float32),
        scratch_types=[
            pltpu.VMEM((_GATHER_CHUNKS, _CHUNK), jnp.int32),
            pltpu.VMEM((_ROWS_PER_W, 2 * _DIM), jnp.float32),
            pltpu.VMEM((_ROWS_PER_W, _DIM), jnp.float32),
            pltpu.SemaphoreType.DMA,
        ],
    )(_sc_gather_body)
    return gather(cbpad, idx_w)


@functools.lru_cache(maxsize=1)
def _argmin_weights():
    import numpy as np
    w = np.zeros((_NUM_CODES, 128), np.float32)
    ids = np.arange(_NUM_CODES)
    w[:, 0] = ids // 32
    w[:, 1] = ids % 32
    w[:, 2] = 1.0
    return jnp.asarray(w)


def _row_specs():
    return [
        pl.BlockSpec((_BLOCK, _DIM), lambda i: (i, 0)),
        pl.BlockSpec((_NUM_CODES, _DIM), lambda i: (0, 0)),
        pl.BlockSpec((_NUM_CODES, 128), lambda i: (0, 0)),
    ]


_VEC_SPEC = pl.BlockSpec((1, _NUM_CODES), lambda i: (0, 0))
_SCALAR_SPEC = pl.BlockSpec((1, 1), lambda i: (0, 0))
_IDX_SPEC = pl.BlockSpec((1, 1, _BLOCK), lambda i: (i, 0, 0))


def kernel(z, codebook):
    b, k, d = z.shape
    flat = z.reshape(-1, d)
    f32 = jnp.float32
    w = _argmin_weights()

    idx1, cbpad, cnt1, soft1, msum1 = pl.pallas_call(
        _vq_body_first,
        grid=(_HGRID,),
        in_specs=_row_specs(),
        out_specs=[
            _IDX_SPEC,
            pl.BlockSpec((_NUM_CODES, 2 * _DIM), lambda i: (0, 0)),
            _VEC_SPEC,
            _VEC_SPEC,
            _SCALAR_SPEC,
        ],
        out_shape=[
            jax.ShapeDtypeStruct((_HGRID, 1, _BLOCK), jnp.int32),
            jax.ShapeDtypeStruct((_NUM_CODES, 2 * _DIM), f32),
            jax.ShapeDtypeStruct((1, _NUM_CODES), f32),
            jax.ShapeDtypeStruct((1, _NUM_CODES), f32),
            jax.ShapeDtypeStruct((1, 1), f32),
        ],
        interpret=_INTERPRET,
    )(flat[:_HALF], codebook, w)

    q1 = _sc_gather(cbpad, idx1.reshape(_SC_WORKERS, _GATHER_CHUNKS, _CHUNK))

    idx2, usage2, soft2, loss2, cbl2, cml2, perp2 = pl.pallas_call(
        _vq_body_second,
        grid=(_HGRID,),
        in_specs=_row_specs() + [_VEC_SPEC, _VEC_SPEC, _SCALAR_SPEC],
        out_specs=[
            _IDX_SPEC,
            _VEC_SPEC,
            _VEC_SPEC,
            _SCALAR_SPEC,
            _SCALAR_SPEC,
            _SCALAR_SPEC,
            _SCALAR_SPEC,
        ],
        out_shape=[
            jax.ShapeDtypeStruct((_HGRID, 1, _BLOCK), jnp.int32),
            jax.ShapeDtypeStruct((1, _NUM_CODES), f32),
            jax.ShapeDtypeStruct((1, _NUM_CODES), f32),
            jax.ShapeDtypeStruct((1, 1), f32),
            jax.ShapeDtypeStruct((1, 1), f32),
            jax.ShapeDtypeStruct((1, 1), f32),
            jax.ShapeDtypeStruct((1, 1), f32),
        ],
        interpret=_INTERPRET,
    )(flat[_HALF:], codebook, w, cnt1, soft1, msum1)

    q2 = _sc_gather(cbpad, idx2.reshape(_SC_WORKERS, _GATHER_CHUNKS, _CHUNK))

    indices = jnp.concatenate([idx1.reshape(-1), idx2.reshape(-1)]).reshape(b, k)
    q = jnp.concatenate([q1, q2], axis=0).reshape(b, k, d)
    return (
        q,
        loss2.reshape(()),
        indices,
        cbl2.reshape(()),
        cml2.reshape(()),
        perp2.reshape(()),
        usage2.reshape(-1),
        soft2.reshape(-1),
    )


# speedup vs baseline: 1.2103x; 1.2103x over previous
import functools

import jax
import jax.numpy as jnp
from jax import lax
from jax.experimental import pallas as pl
from jax.experimental.pallas import tpu as pltpu
from jax.experimental.pallas import tpu_sc as plsc

_NUM_CODES = 1024
_DIM = 64
_ROWS = 9216
_BLOCK = 3072
_GRID = _ROWS // _BLOCK
_COMMIT_W = 0.25

_SC_CORES = 2
_SC_SUBCORES = 16
_SC_WORKERS = _SC_CORES * _SC_SUBCORES
_ROWS_PER_W = _ROWS // _SC_WORKERS
_GATHER_CHUNKS = 3
_CHUNK = _ROWS_PER_W // _GATHER_CHUNKS

_INTERPRET = False


def _vq_body(z_ref, cb_ref, w_ref,
             idx_ref, cbpad_ref, usage_ref, soft_ref,
             loss_ref, cbl_ref, cml_ref, perp_ref):
    step = pl.program_id(0)
    zb = z_ref[...]
    cb = cb_ref[...]

    znorm = jnp.sum(zb * zb, axis=1, keepdims=True)
    cnorm = jnp.sum(cb * cb, axis=1)[None, :]
    cross = lax.dot_general(zb, cb, (((1,), (1,)), ((), ())))
    d = znorm - 2.0 * cross + cnorm

    dmin = jnp.min(d, axis=1, keepdims=True)
    mask = d == dmin
    maskf = mask.astype(jnp.float32)

    mm = lax.dot_general(maskf, w_ref[...], (((1,), (0,)), ((), ())))
    idx_f = mm[:, 0:1] * 32.0 + mm[:, 1:2]
    idx_ref[...] = idx_f.astype(jnp.int32).reshape(-1)
    anytie = jnp.max(mm[:, 2:3]) > 1.5

    e = jnp.exp(dmin - d)
    ones_n = jnp.ones((_NUM_CODES, 1), jnp.float32)
    s = lax.dot_general(e, ones_n, (((1,), (0,)), ((), ())))
    r = 1.0 / s
    block_soft = lax.dot_general(r, e, (((0,), (0,)), ((), ())))
    ones_b = jnp.ones((zb.shape[0], 1), jnp.float32)
    block_counts = lax.dot_general(ones_b, maskf,
                                   (((0,), (0,)), ((), ())))

    block_msum = jnp.sum(dmin, axis=0, keepdims=True)

    @pl.when(step == 0)
    def _init():
        cbpad_ref[...] = jnp.concatenate([cb, cb], axis=1)
        usage_ref[...] = block_counts
        soft_ref[...] = block_soft
        loss_ref[...] = block_msum

    @pl.when(step > 0)
    def _accum():
        usage_ref[...] += block_counts
        soft_ref[...] += block_soft
        loss_ref[...] += block_msum

    @pl.when(anytie)
    def _tie_fix():
        iota = lax.broadcasted_iota(jnp.int32, d.shape, 1)
        idx_e = jnp.min(jnp.where(mask, iota, _NUM_CODES), axis=1)
        idx_ref[...] = idx_e
        onehot = (iota == idx_e[:, None]).astype(jnp.float32)
        usage_ref[...] += jnp.sum(onehot - maskf, axis=0)[None, :]

    @pl.when(step == _GRID - 1)
    def _finalize():
        counts = usage_ref[...]
        total = jnp.sum(counts, axis=1, keepdims=True)
        u = counts / jnp.maximum(total, 1.0)
        usage_ref[...] = u
        soft_ref[...] = soft_ref[...] / float(_ROWS)
        mse = loss_ref[...] / float(_ROWS * _DIM)
        cbl_ref[...] = mse
        cml_ref[...] = mse
        loss_ref[...] = mse + _COMMIT_W * mse
        ent = jnp.sum(u * jnp.log(u + 1e-08), axis=1, keepdims=True)
        perp_ref[...] = jnp.exp(-ent)


def _sc_gather_body(cbpad_hbm, idx_hbm, out_hbm, idx_v, rows_v, comp_v, sem):
    wid = lax.axis_index("s") * _SC_CORES + lax.axis_index("c")
    base = wid * _ROWS_PER_W
    for k in range(_GATHER_CHUNKS):
        pltpu.sync_copy(idx_hbm.at[pl.ds(base + k * _CHUNK, _CHUNK)],
                        idx_v.at[k])
    copies = [
        pltpu.async_copy(cbpad_hbm.at[idx_v.at[k]],
                         rows_v.at[pl.ds(k * _CHUNK, _CHUNK)], sem)
        for k in range(_GATHER_CHUNKS)
    ]
    for c in copies:
        c.wait()

    def _row(r, carry):
        for j in range(_DIM // 16):
            comp_v[r, pl.ds(16 * j, 16)] = rows_v[r, pl.ds(16 * j, 16)]
        return carry

    lax.fori_loop(0, _ROWS_PER_W, _row, 0, unroll=2)
    pltpu.sync_copy(comp_v, out_hbm.at[pl.ds(base, _ROWS_PER_W)])


def _sc_gather(cbpad, idx_w):
    gather = functools.partial(
        pl.kernel,
        mesh=plsc.VectorSubcoreMesh(core_axis_name="c", subcore_axis_name="s"),
        out_type=jax.ShapeDtypeStruct((_ROWS, _DIM), jnp.float32),
        scratch_types=[
            pltpu.VMEM((_GATHER_CHUNKS, _CHUNK), jnp.int32),
            pltpu.VMEM((_ROWS_PER_W, 2 * _DIM), jnp.float32),
            pltpu.VMEM((_ROWS_PER_W, _DIM), jnp.float32),
            pltpu.SemaphoreType.DMA,
        ],
    )(_sc_gather_body)
    return gather(cbpad, idx_w)


@functools.lru_cache(maxsize=1)
def _argmin_weights():
    import numpy as np
    w = np.zeros((_NUM_CODES, 128), np.float32)
    ids = np.arange(_NUM_CODES)
    w[:, 0] = ids // 32
    w[:, 1] = ids % 32
    w[:, 2] = 1.0
    return jnp.asarray(w)


def kernel(z, codebook):
    b, k, d = z.shape
    flat = z.reshape(-1, d)
    f32 = jnp.float32
    outs = pl.pallas_call(
        _vq_body,
        grid=(_GRID,),
        in_specs=[
            pl.BlockSpec((_BLOCK, _DIM), lambda i: (i, 0)),
            pl.BlockSpec((_NUM_CODES, _DIM), lambda i: (0, 0)),
            pl.BlockSpec((_NUM_CODES, 128), lambda i: (0, 0)),
        ],
        out_specs=[
            pl.BlockSpec((_BLOCK,), lambda i: (i,)),
            pl.BlockSpec((_NUM_CODES, 2 * _DIM), lambda i: (0, 0)),
            pl.BlockSpec((1, _NUM_CODES), lambda i: (0, 0)),
            pl.BlockSpec((1, _NUM_CODES), lambda i: (0, 0)),
            pl.BlockSpec((1, 1), lambda i: (0, 0)),
            pl.BlockSpec((1, 1), lambda i: (0, 0)),
            pl.BlockSpec((1, 1), lambda i: (0, 0)),
            pl.BlockSpec((1, 1), lambda i: (0, 0)),
        ],
        out_shape=[
            jax.ShapeDtypeStruct((_ROWS,), jnp.int32),
            jax.ShapeDtypeStruct((_NUM_CODES, 2 * _DIM), f32),
            jax.ShapeDtypeStruct((1, _NUM_CODES), f32),
            jax.ShapeDtypeStruct((1, _NUM_CODES), f32),
            jax.ShapeDtypeStruct((1, 1), f32),
            jax.ShapeDtypeStruct((1, 1), f32),
            jax.ShapeDtypeStruct((1, 1), f32),
            jax.ShapeDtypeStruct((1, 1), f32),
        ],
        interpret=_INTERPRET,
    )(flat, codebook, _argmin_weights())
    idx1d, cbpad, usage2, soft2, loss2, cbl2, cml2, perp2 = outs
    q = _sc_gather(cbpad, idx1d)
    return (
        q.reshape(b, k, d),
        loss2.reshape(()),
        idx1d.reshape(b, k),
        cbl2.reshape(()),
        cml2.reshape(()),
        perp2.reshape(()),
        usage2.reshape(-1),
        soft2.reshape(-1),
    )


# speedup vs baseline: 1.2367x; 1.0218x over previous
import functools

import jax
import jax.numpy as jnp
from jax import lax
from jax.experimental import pallas as pl
from jax.experimental.pallas import tpu as pltpu
from jax.experimental.pallas import tpu_sc as plsc

_NUM_CODES = 1024
_DIM = 64
_ROWS = 9216
_BLOCK = 3072
_GRID = _ROWS // _BLOCK
_COMMIT_W = 0.25

_SC_CORES = 2
_SC_SUBCORES = 16
_SC_WORKERS = _SC_CORES * _SC_SUBCORES
_ROWS_PER_W = _ROWS // _SC_WORKERS
_GATHER_CHUNKS = 3
_CHUNK = _ROWS_PER_W // _GATHER_CHUNKS

_INTERPRET = False


def _vq_body(z_ref, cb_ref, w_ref,
             idx_ref, cbpad_ref, usage_ref, soft_ref,
             loss_ref, cbl_ref, cml_ref, perp_ref):
    step = pl.program_id(0)
    zb = z_ref[...]
    cb = cb_ref[...]

    znorm = jnp.sum(zb * zb, axis=1, keepdims=True)
    cnorm = jnp.sum(cb * cb, axis=1)[None, :]
    cross = lax.dot_general(zb, cb, (((1,), (1,)), ((), ())))
    d = znorm - 2.0 * cross + cnorm

    dmin = jnp.min(d, axis=1, keepdims=True)
    mask = d == dmin
    maskf = mask.astype(jnp.float32)

    mm = lax.dot_general(maskf, w_ref[...], (((1,), (0,)), ((), ())))
    idx_f = mm[:, 0:1] * 32.0 + mm[:, 1:2]
    idx_ref[...] = idx_f.astype(jnp.int32).reshape(-1)
    anytie = jnp.max(mm[:, 2:3]) > 1.5

    e = jnp.exp(dmin - d)
    ones_n = jnp.ones((_NUM_CODES, 1), jnp.float32)
    s = lax.dot_general(e, ones_n, (((1,), (0,)), ((), ())))
    r = 1.0 / s
    block_soft = lax.dot_general(r, e, (((0,), (0,)), ((), ())))
    ones_b = jnp.ones((zb.shape[0], 1), jnp.float32)
    block_counts = lax.dot_general(ones_b, maskf,
                                   (((0,), (0,)), ((), ())))

    block_msum = jnp.sum(dmin, axis=0, keepdims=True)

    @pl.when(step == 0)
    def _init():
        cbpad_ref[...] = jnp.concatenate([cb, cb], axis=1)
        usage_ref[...] = block_counts
        soft_ref[...] = block_soft
        loss_ref[...] = block_msum

    @pl.when(step > 0)
    def _accum():
        usage_ref[...] += block_counts
        soft_ref[...] += block_soft
        loss_ref[...] += block_msum

    @pl.when(anytie)
    def _tie_fix():
        iota = lax.broadcasted_iota(jnp.int32, d.shape, 1)
        idx_e = jnp.min(jnp.where(mask, iota, _NUM_CODES), axis=1)
        idx_ref[...] = idx_e
        onehot = (iota == idx_e[:, None]).astype(jnp.float32)
        usage_ref[...] += jnp.sum(onehot - maskf, axis=0)[None, :]

    @pl.when(step == _GRID - 1)
    def _finalize():
        counts = usage_ref[...]
        total = jnp.sum(counts, axis=1, keepdims=True)
        u = counts / jnp.maximum(total, 1.0)
        usage_ref[...] = u
        soft_ref[...] = soft_ref[...] / float(_ROWS)
        mse = loss_ref[...] / float(_ROWS * _DIM)
        cbl_ref[...] = mse
        cml_ref[...] = mse
        loss_ref[...] = mse + _COMMIT_W * mse
        ent = jnp.sum(u * jnp.log(u + 1e-08), axis=1, keepdims=True)
        perp_ref[...] = jnp.exp(-ent)


def _sc_gather_body(cbpad_hbm, idx_hbm, out_hbm, idx_v, rows_v, sem):
    wid = lax.axis_index("s") * _SC_CORES + lax.axis_index("c")
    base = wid * _ROWS_PER_W
    for k in range(_GATHER_CHUNKS):
        pltpu.sync_copy(idx_hbm.at[pl.ds(base + k * _CHUNK, _CHUNK)],
                        idx_v.at[k])
    copies = [
        pltpu.async_copy(cbpad_hbm.at[idx_v.at[k]],
                         rows_v.at[pl.ds(k * _CHUNK, _CHUNK)], sem)
        for k in range(_GATHER_CHUNKS)
    ]
    for c in copies:
        c.wait()

    pltpu.sync_copy(rows_v, out_hbm.at[pl.ds(base, _ROWS_PER_W)])


def _sc_gather(cbpad, idx_w):
    gather = functools.partial(
        pl.kernel,
        mesh=plsc.VectorSubcoreMesh(core_axis_name="c", subcore_axis_name="s"),
        out_type=jax.ShapeDtypeStruct((_ROWS, 2 * _DIM), jnp.float32),
        scratch_types=[
            pltpu.VMEM((_GATHER_CHUNKS, _CHUNK), jnp.int32),
            pltpu.VMEM((_ROWS_PER_W, 2 * _DIM), jnp.float32),
            pltpu.SemaphoreType.DMA,
        ],
    )(_sc_gather_body)
    return gather(cbpad, idx_w)


@functools.lru_cache(maxsize=1)
def _argmin_weights():
    import numpy as np
    w = np.zeros((_NUM_CODES, 128), np.float32)
    ids = np.arange(_NUM_CODES)
    w[:, 0] = ids // 32
    w[:, 1] = ids % 32
    w[:, 2] = 1.0
    return jnp.asarray(w)


def kernel(z, codebook):
    b, k, d = z.shape
    flat = z.reshape(-1, d)
    f32 = jnp.float32
    outs = pl.pallas_call(
        _vq_body,
        grid=(_GRID,),
        in_specs=[
            pl.BlockSpec((_BLOCK, _DIM), lambda i: (i, 0)),
            pl.BlockSpec((_NUM_CODES, _DIM), lambda i: (0, 0)),
            pl.BlockSpec((_NUM_CODES, 128), lambda i: (0, 0)),
        ],
        out_specs=[
            pl.BlockSpec((_BLOCK,), lambda i: (i,)),
            pl.BlockSpec((_NUM_CODES, 2 * _DIM), lambda i: (0, 0)),
            pl.BlockSpec((1, _NUM_CODES), lambda i: (0, 0)),
            pl.BlockSpec((1, _NUM_CODES), lambda i: (0, 0)),
            pl.BlockSpec((1, 1), lambda i: (0, 0)),
            pl.BlockSpec((1, 1), lambda i: (0, 0)),
            pl.BlockSpec((1, 1), lambda i: (0, 0)),
            pl.BlockSpec((1, 1), lambda i: (0, 0)),
        ],
        out_shape=[
            jax.ShapeDtypeStruct((_ROWS,), jnp.int32),
            jax.ShapeDtypeStruct((_NUM_CODES, 2 * _DIM), f32),
            jax.ShapeDtypeStruct((1, _NUM_CODES), f32),
            jax.ShapeDtypeStruct((1, _NUM_CODES), f32),
            jax.ShapeDtypeStruct((1, 1), f32),
            jax.ShapeDtypeStruct((1, 1), f32),
            jax.ShapeDtypeStruct((1, 1), f32),
            jax.ShapeDtypeStruct((1, 1), f32),
        ],
        interpret=_INTERPRET,
    )(flat, codebook, _argmin_weights())
    idx1d, cbpad, usage2, soft2, loss2, cbl2, cml2, perp2 = outs
    qpad = _sc_gather(cbpad, idx1d)
    return (
        qpad[:, :_DIM].reshape(b, k, d),
        loss2.reshape(()),
        idx1d.reshape(b, k),
        cbl2.reshape(()),
        cml2.reshape(()),
        perp2.reshape(()),
        usage2.reshape(-1),
        soft2.reshape(-1),
    )


# speedup vs baseline: 1.2392x; 1.0020x over previous
import functools

import jax
import jax.numpy as jnp
from jax import lax
from jax.experimental import pallas as pl
from jax.experimental.pallas import tpu as pltpu
from jax.experimental.pallas import tpu_sc as plsc

_NUM_CODES = 1024
_DIM = 64
_ROWS = 9216
_BLOCK = 3072
_GRID = _ROWS // _BLOCK
_COMMIT_W = 0.25

_SC_CORES = 2
_SC_SUBCORES = 16
_SC_WORKERS = _SC_CORES * _SC_SUBCORES
_ROWS_PER_W = _ROWS // _SC_WORKERS
_GATHER_CHUNKS = 3
_CHUNK = _ROWS_PER_W // _GATHER_CHUNKS


def _vq_body(z_ref, cb_ref, w_ref,
             idx_ref, cbpad_ref, usage_ref, soft_ref,
             loss_ref, cbl_ref, cml_ref, perp_ref):
    step = pl.program_id(0)
    zb = z_ref[...]
    cb = cb_ref[...]

    znorm = jnp.sum(zb * zb, axis=1, keepdims=True)
    cnorm = jnp.sum(cb * cb, axis=1)[None, :]
    cross = lax.dot_general(zb, cb, (((1,), (1,)), ((), ())))
    d = znorm - 2.0 * cross + cnorm

    dmin = jnp.min(d, axis=1, keepdims=True)
    mask = d == dmin
    maskf = mask.astype(jnp.float32)

    mm = lax.dot_general(maskf, w_ref[...], (((1,), (0,)), ((), ())))
    idx_f = mm[:, 0:1] * 32.0 + mm[:, 1:2]
    idx_ref[...] = idx_f.astype(jnp.int32).reshape(-1)
    anytie = jnp.max(mm[:, 2:3]) > 1.5

    e = jnp.exp(dmin - d)
    ones_n = jnp.ones((_NUM_CODES, 1), jnp.float32)
    s = lax.dot_general(e, ones_n, (((1,), (0,)), ((), ())))
    r = 1.0 / s
    block_soft = lax.dot_general(r, e, (((0,), (0,)), ((), ())))
    ones_b = jnp.ones((zb.shape[0], 1), jnp.float32)
    block_counts = lax.dot_general(ones_b, maskf,
                                   (((0,), (0,)), ((), ())))

    block_msum = jnp.sum(dmin, axis=0, keepdims=True)

    @pl.when(step == 0)
    def _init():
        cbpad_ref[...] = jnp.concatenate([cb, cb], axis=1)
        usage_ref[...] = block_counts
        soft_ref[...] = block_soft
        loss_ref[...] = block_msum

    @pl.when(step > 0)
    def _accum():
        usage_ref[...] += block_counts
        soft_ref[...] += block_soft
        loss_ref[...] += block_msum

    @pl.when(anytie)
    def _tie_fix():
        iota = lax.broadcasted_iota(jnp.int32, d.shape, 1)
        idx_e = jnp.min(jnp.where(mask, iota, _NUM_CODES), axis=1)
        idx_ref[...] = idx_e
        onehot = (iota == idx_e[:, None]).astype(jnp.float32)
        usage_ref[...] += jnp.sum(onehot - maskf, axis=0)[None, :]

    @pl.when(step == _GRID - 1)
    def _finalize():
        counts = usage_ref[...]
        total = jnp.sum(counts, axis=1, keepdims=True)
        u = counts / jnp.maximum(total, 1.0)
        usage_ref[...] = u
        soft_ref[...] = soft_ref[...] / float(_ROWS)
        mse = loss_ref[...] / float(_ROWS * _DIM)
        cbl_ref[...] = mse
        cml_ref[...] = mse
        loss_ref[...] = mse + _COMMIT_W * mse
        ent = jnp.sum(u * jnp.log(u + 1e-08), axis=1, keepdims=True)
        perp_ref[...] = jnp.exp(-ent)


def _sc_gather_body(cbpad_hbm, idx_hbm, out_hbm, idx_v, rows_v, sem):
    wid = lax.axis_index("s") * _SC_CORES + lax.axis_index("c")
    base = wid * _ROWS_PER_W
    for k in range(_GATHER_CHUNKS):
        pltpu.sync_copy(idx_hbm.at[pl.ds(base + k * _CHUNK, _CHUNK)],
                        idx_v.at[k])
    copies = [
        pltpu.async_copy(cbpad_hbm.at[idx_v.at[k]],
                         rows_v.at[pl.ds(k * _CHUNK, _CHUNK)], sem)
        for k in range(_GATHER_CHUNKS)
    ]
    for c in copies:
        c.wait()

    pltpu.sync_copy(rows_v, out_hbm.at[pl.ds(base, _ROWS_PER_W)])


def _sc_gather(cbpad, idx_w):
    gather = functools.partial(
        pl.kernel,
        mesh=plsc.VectorSubcoreMesh(core_axis_name="c", subcore_axis_name="s"),
        out_type=jax.ShapeDtypeStruct((_ROWS, 2 * _DIM), jnp.float32),
        scratch_types=[
            pltpu.VMEM((_GATHER_CHUNKS, _CHUNK), jnp.int32),
            pltpu.VMEM((_ROWS_PER_W, 2 * _DIM), jnp.float32),
            pltpu.SemaphoreType.DMA,
        ],
    )(_sc_gather_body)
    return gather(cbpad, idx_w)


@functools.lru_cache(maxsize=1)
def _argmin_weights():
    import numpy as np
    w = np.zeros((_NUM_CODES, 128), np.float32)
    ids = np.arange(_NUM_CODES)
    w[:, 0] = ids // 32
    w[:, 1] = ids % 32
    w[:, 2] = 1.0
    return jnp.asarray(w)


def kernel(z, codebook):
    b, k, d = z.shape
    flat = z.reshape(-1, d)
    f32 = jnp.float32
    outs = pl.pallas_call(
        _vq_body,
        grid=(_GRID,),
        in_specs=[
            pl.BlockSpec((_BLOCK, _DIM), lambda i: (i, 0)),
            pl.BlockSpec((_NUM_CODES, _DIM), lambda i: (0, 0)),
            pl.BlockSpec((_NUM_CODES, 128), lambda i: (0, 0)),
        ],
        out_specs=[
            pl.BlockSpec((_BLOCK,), lambda i: (i,)),
            pl.BlockSpec((_NUM_CODES, 2 * _DIM), lambda i: (0, 0)),
            pl.BlockSpec((1, _NUM_CODES), lambda i: (0, 0)),
            pl.BlockSpec((1, _NUM_CODES), lambda i: (0, 0)),
            pl.BlockSpec((1, 1), lambda i: (0, 0)),
            pl.BlockSpec((1, 1), lambda i: (0, 0)),
            pl.BlockSpec((1, 1), lambda i: (0, 0)),
            pl.BlockSpec((1, 1), lambda i: (0, 0)),
        ],
        out_shape=[
            jax.ShapeDtypeStruct((_ROWS,), jnp.int32),
            jax.ShapeDtypeStruct((_NUM_CODES, 2 * _DIM), f32),
            jax.ShapeDtypeStruct((1, _NUM_CODES), f32),
            jax.ShapeDtypeStruct((1, _NUM_CODES), f32),
            jax.ShapeDtypeStruct((1, 1), f32),
            jax.ShapeDtypeStruct((1, 1), f32),
            jax.ShapeDtypeStruct((1, 1), f32),
            jax.ShapeDtypeStruct((1, 1), f32),
        ],
    )(flat, codebook, _argmin_weights())
    idx1d, cbpad, usage2, soft2, loss2, cbl2, cml2, perp2 = outs
    qpad = _sc_gather(cbpad, idx1d)
    return (
        qpad[:, :_DIM].reshape(b, k, d),
        loss2.reshape(()),
        idx1d.reshape(b, k),
        cbl2.reshape(()),
        cml2.reshape(()),
        perp2.reshape(()),
        usage2.reshape(-1),
        soft2.reshape(-1),
    )
